# Initial kernel scaffold; baseline (speedup 1.0000x reference)
import jax
import jax.numpy as jnp
from jax.experimental import pallas as pl


def kernel(voxels, voxel_num_points, point_voxel_ids, W):
    raise NotImplementedError("write your pallas kernel here")


# speedup vs baseline: 3.0852x; 3.0852x over previous
import jax
import jax.numpy as jnp
from jax import lax
from jax.experimental import pallas as pl
from jax.experimental.pallas import tpu as pltpu

_pc = pl.pallas_call

VPR = 32
M_BLK = 640
BJ1 = 512


def _table_body(vr_ref, cnt_ref, tbl_ref):
    x = vr_ref[...]
    s = (x[:, 0:128] + x[:, 128:256] + x[:, 256:384]
         + x[:, 384:512] + x[:, 512:640])
    inv = 1.0 / jnp.maximum(cnt_ref[...], 1).astype(jnp.float32)
    tbl_ref[...] = s * inv


def _gather_body(jrow_ref, ids_ref, wt_ref, tbl_ref, out_ref, x_scr):
    m = out_ref.shape[0]
    idsv = ids_ref[0]
    gior = lax.broadcasted_iota(jnp.int32, (VPR, m), 0)
    gb = jnp.broadcast_to(idsv & (VPR - 1), (VPR, m))
    vb = jnp.broadcast_to(idsv, (VPR, m))
    e = jnp.where((gb == gior) & (vb >= 0), 1.0, 0.0)
    lio = lax.broadcasted_iota(jnp.int32, (VPR, 128), 1) >> 2
    sio = lax.broadcasted_iota(jnp.int32, (VPR, 128), 0)
    smat = jnp.where(lio == sio, 1.0, 0.0)
    g4 = lax.dot_general(e, smat, (((0,), (0,)), ((), ())),
                         preferred_element_type=jnp.float32)
    for mi in range(m):
        j = jrow_ref[0, 0, mi]
        base = pl.multiple_of((j >> 3) << 3, 8)
        c = tbl_ref[pl.ds(base, 8), :]
        r = pltpu.roll(c, (mi - j) & 7, axis=0)
        s = mi & 7
        x_scr[mi:mi + 1, :] = r[s:s + 1, :]
    y = x_scr[...] * g4
    out_ref[...] = jnp.maximum(
        jnp.dot(y, wt_ref[...], preferred_element_type=jnp.float32), 0.0)


def kernel(voxels, voxel_num_points, point_voxel_ids, W):
    nv = voxels.shape[0]
    npts = point_voxel_ids.shape[0]
    rows = -(-nv // VPR)
    rows = -(-rows // BJ1) * BJ1
    nvp = rows * VPR
    f32 = jnp.float32

    vpad = jnp.pad(voxels, ((0, nvp - nv), (0, 0), (0, 0)))
    vr = (vpad.reshape(rows, VPR, 5, 4).transpose(0, 2, 1, 3)
          .reshape(rows, 640))
    cntp = jnp.repeat(
        jnp.pad(voxel_num_points, (0, nvp - nv), constant_values=1), 4
    ).reshape(rows, 128)

    tbl = _pc(
        _table_body,
        grid=(rows // BJ1,),
        in_specs=[
            pl.BlockSpec((BJ1, 640), lambda i: (i, 0)),
            pl.BlockSpec((BJ1, 128), lambda i: (i, 0)),
        ],
        out_specs=pl.BlockSpec((BJ1, 128), lambda i: (i, 0)),
        out_shape=jax.ShapeDtypeStruct((rows, 128), f32),
        compiler_params=pltpu.CompilerParams(
            dimension_semantics=("parallel",)),
    )(vr, cntp)

    m = M_BLK if npts % M_BLK == 0 else npts
    nb = npts // m
    safe = jnp.where(point_voxel_ids >= 0, point_voxel_ids, 0)
    jrow = (safe >> 5).astype(jnp.int32).reshape(nb, 1, m)
    idsr = point_voxel_ids.reshape(nb, 1, m)
    wt = jnp.tile(W, (VPR, 1))

    out = _pc(
        _gather_body,
        grid=(nb,),
        in_specs=[
            pl.BlockSpec((1, 1, m), lambda i: (i, 0, 0),
                         memory_space=pltpu.SMEM),
            pl.BlockSpec((1, 1, m), lambda i: (i, 0, 0)),
            pl.BlockSpec((128, 32), lambda i: (0, 0)),
            pl.BlockSpec(tbl.shape, lambda i: (0, 0)),
        ],
        out_specs=pl.BlockSpec((m, 32), lambda i: (i, 0)),
        out_shape=jax.ShapeDtypeStruct((npts, 32), f32),
        scratch_shapes=[pltpu.VMEM((m, 128), f32)],
        compiler_params=pltpu.CompilerParams(
            dimension_semantics=("parallel",)),
    )(jrow, idsr, wt, tbl)
    return out
